# baseline (speedup 1.0000x reference)
import jax
import jax.numpy as jnp
from jax.experimental import pallas as pl
from jax.experimental.pallas import tpu as pltpu
from jax.sharding import Mesh, PartitionSpec as P
from jax.experimental.shard_map import shard_map

N_F = 1024
N_K = 4096
B_TILE = 512


def _prep_kernel(a_ref, an_ref):
    a = a_ref[...]
    inv = jax.lax.rsqrt(jnp.sum(a * a, axis=0, keepdims=True))
    an_ref[...] = (a * inv).astype(jnp.bfloat16)


def _fused_kernel(x_ref, an_ref, b_ref, inner_ref, out_ref):
    xb = x_ref[...].astype(jnp.bfloat16)
    inner = jnp.dot(xb, an_ref[...], preferred_element_type=jnp.float32)
    inner_ref[...] = inner
    out_ref[...] = jnp.dot(inner.astype(jnp.bfloat16), b_ref[...],
                           preferred_element_type=jnp.float32)


def _per_shard(x, A, B):
    batch = x.shape[0]
    An = pl.pallas_call(
        _prep_kernel,
        out_shape=jax.ShapeDtypeStruct((N_F, N_K), jnp.bfloat16),
    )(A)
    Bb = B.astype(jnp.bfloat16)
    grid = (batch // B_TILE,)
    inner, out = pl.pallas_call(
        _fused_kernel,
        grid=grid,
        in_specs=[
            pl.BlockSpec((B_TILE, N_F), lambda i: (i, 0)),
            pl.BlockSpec((N_F, N_K), lambda i: (0, 0)),
            pl.BlockSpec((N_K, N_F), lambda i: (0, 0)),
        ],
        out_specs=[
            pl.BlockSpec((B_TILE, N_K), lambda i: (i, 0)),
            pl.BlockSpec((B_TILE, N_F), lambda i: (i, 0)),
        ],
        out_shape=[
            jax.ShapeDtypeStruct((batch, N_K), jnp.float32),
            jax.ShapeDtypeStruct((batch, N_F), jnp.float32),
        ],
        compiler_params=pltpu.CompilerParams(
            dimension_semantics=("arbitrary",),
        ),
    )(x, An, Bb)
    return (out, inner)


def kernel(x, A, B):
    devs = jax.devices()
    ndev = 2 if len(devs) >= 2 and x.shape[0] % 2 == 0 else 1
    if ndev == 1:
        return _per_shard(x, A, B)
    mesh = Mesh(devs[:ndev], ("d",))
    fn = shard_map(
        _per_shard,
        mesh=mesh,
        in_specs=(P("d", None), P(None, None), P(None, None)),
        out_specs=(P("d", None), P("d", None)),
        check_rep=False,
    )
    return fn(x, A, B)


# speedup vs baseline: 4.6761x; 4.6761x over previous
import functools

import jax
import jax.numpy as jnp
from jax.experimental import pallas as pl
from jax.experimental.pallas import tpu as pltpu

N_F = 1024
N_K = 4096
B_TILE = 512


def _prep_kernel(a_ref, b_ref, an_ref, bb_ref):
    a = a_ref[...]
    inv = jax.lax.rsqrt(jnp.sum(a * a, axis=0, keepdims=True))
    an_ref[...] = (a * inv).astype(jnp.bfloat16)
    bb_ref[...] = b_ref[...].astype(jnp.bfloat16)


def _fused_kernel(x_ref, an_ref, b_ref, inner_ref, out_ref):
    xb = x_ref[...].astype(jnp.bfloat16)
    inner = jnp.dot(xb, an_ref[...], preferred_element_type=jnp.float32)
    inner_ref[...] = inner
    out_ref[...] = jnp.dot(inner.astype(jnp.bfloat16), b_ref[...],
                           preferred_element_type=jnp.float32)


@functools.partial(jax.jit, static_argnums=())
def kernel(x, A, B):
    batch = x.shape[0]
    P_CH = 8
    An, Bb = pl.pallas_call(
        _prep_kernel,
        grid=(P_CH,),
        in_specs=[
            pl.BlockSpec((N_F, N_K // P_CH), lambda j: (0, j)),
            pl.BlockSpec((N_K // P_CH, N_F), lambda j: (j, 0)),
        ],
        out_specs=[
            pl.BlockSpec((N_F, N_K // P_CH), lambda j: (0, j)),
            pl.BlockSpec((N_K // P_CH, N_F), lambda j: (j, 0)),
        ],
        out_shape=[
            jax.ShapeDtypeStruct((N_F, N_K), jnp.bfloat16),
            jax.ShapeDtypeStruct((N_K, N_F), jnp.bfloat16),
        ],
        compiler_params=pltpu.CompilerParams(
            dimension_semantics=("arbitrary",),
        ),
    )(A, B)
    grid = (batch // B_TILE,)
    inner, out = pl.pallas_call(
        _fused_kernel,
        grid=grid,
        in_specs=[
            pl.BlockSpec((B_TILE, N_F), lambda i: (i, 0)),
            pl.BlockSpec((N_F, N_K), lambda i: (0, 0)),
            pl.BlockSpec((N_K, N_F), lambda i: (0, 0)),
        ],
        out_specs=[
            pl.BlockSpec((B_TILE, N_K), lambda i: (i, 0)),
            pl.BlockSpec((B_TILE, N_F), lambda i: (i, 0)),
        ],
        out_shape=[
            jax.ShapeDtypeStruct((batch, N_K), jnp.float32),
            jax.ShapeDtypeStruct((batch, N_F), jnp.float32),
        ],
        compiler_params=pltpu.CompilerParams(
            dimension_semantics=("parallel",),
        ),
    )(x, An, Bb)
    return (out, inner)


# speedup vs baseline: 5.1257x; 1.0961x over previous
import jax
import jax.numpy as jnp
from jax.experimental import pallas as pl
from jax.experimental.pallas import tpu as pltpu

N_F = 1024
N_K = 4096
B_TILE = 512
P_CH = 8
KC = N_K // P_CH


def _mega_kernel(x_ref, a_ref, b_ref, inner_ref, out_ref, an_ref, bb_ref):
    i = pl.program_id(0)

    @pl.when(i < P_CH)
    def _prep():
        a = a_ref[...]
        inv = jax.lax.rsqrt(jnp.sum(a * a, axis=0, keepdims=True))
        an_ref[:, pl.ds(i * KC, KC)] = (a * inv).astype(jnp.bfloat16)
        bb_ref[pl.ds(i * KC, KC), :] = b_ref[...].astype(jnp.bfloat16)

    @pl.when(i >= P_CH)
    def _fused():
        xb = x_ref[...].astype(jnp.bfloat16)
        inner = jnp.dot(xb, an_ref[...], preferred_element_type=jnp.float32)
        inner_ref[...] = inner
        out_ref[...] = jnp.dot(inner.astype(jnp.bfloat16), bb_ref[...],
                               preferred_element_type=jnp.float32)


def kernel(x, A, B):
    batch = x.shape[0]
    nb = batch // B_TILE
    grid = (P_CH + nb,)

    def x_idx(i):
        j = jnp.maximum(i - P_CH, 0)
        return (j, 0)

    def a_idx(i):
        return (0, jnp.minimum(i, P_CH - 1))

    def b_idx(i):
        return (jnp.minimum(i, P_CH - 1), 0)

    inner, out = pl.pallas_call(
        _mega_kernel,
        grid=grid,
        in_specs=[
            pl.BlockSpec((B_TILE, N_F), x_idx),
            pl.BlockSpec((N_F, KC), a_idx),
            pl.BlockSpec((KC, N_F), b_idx),
        ],
        out_specs=[
            pl.BlockSpec((B_TILE, N_K), x_idx),
            pl.BlockSpec((B_TILE, N_F), x_idx),
        ],
        out_shape=[
            jax.ShapeDtypeStruct((batch, N_K), jnp.float32),
            jax.ShapeDtypeStruct((batch, N_F), jnp.float32),
        ],
        scratch_shapes=[
            pltpu.VMEM((N_F, N_K), jnp.bfloat16),
            pltpu.VMEM((N_K, N_F), jnp.bfloat16),
        ],
        compiler_params=pltpu.CompilerParams(
            dimension_semantics=("arbitrary",),
        ),
    )(x, A, B)
    return (out, inner)


# speedup vs baseline: 5.3207x; 1.0381x over previous
import jax
import jax.numpy as jnp
from jax.experimental import pallas as pl
from jax.experimental.pallas import tpu as pltpu

N_F = 1024
N_K = 4096
B_TILE = 512
P_CH = 8
KC = N_K // P_CH


def _mega_kernel(x_ref, a_ref, b_ref, inner_ref, out_ref, an_ref, bb_ref):
    i = pl.program_id(0)

    @pl.when(i < P_CH)
    def _prep():
        a = a_ref[...]
        inv = jax.lax.rsqrt(jnp.sum(a * a, axis=0, keepdims=True))
        an_c = (a * inv).astype(jnp.bfloat16)
        bb_c = b_ref[...].astype(jnp.bfloat16)
        an_ref[:, pl.ds(i * KC, KC)] = an_c
        bb_ref[pl.ds(i * KC, KC), :] = bb_c
        xb = x_ref[...].astype(jnp.bfloat16)
        ic = jnp.dot(xb, an_c, preferred_element_type=jnp.float32)
        inner_ref[:, pl.ds(i * KC, KC)] = ic
        part = jnp.dot(ic.astype(jnp.bfloat16), bb_c,
                       preferred_element_type=jnp.float32)

        @pl.when(i == 0)
        def _():
            out_ref[...] = part

        @pl.when(i > 0)
        def _():
            out_ref[...] += part

    @pl.when(i >= P_CH)
    def _fused():
        xb = x_ref[...].astype(jnp.bfloat16)
        inner = jnp.dot(xb, an_ref[...], preferred_element_type=jnp.float32)
        inner_ref[...] = inner
        out_ref[...] = jnp.dot(inner.astype(jnp.bfloat16), bb_ref[...],
                               preferred_element_type=jnp.float32)


def kernel(x, A, B):
    batch = x.shape[0]
    nb = batch // B_TILE
    grid = (P_CH + nb - 1,)

    def x_idx(i):
        return (jnp.maximum(i - (P_CH - 1), 0), 0)

    def a_idx(i):
        return (0, jnp.minimum(i, P_CH - 1))

    def b_idx(i):
        return (jnp.minimum(i, P_CH - 1), 0)

    inner, out = pl.pallas_call(
        _mega_kernel,
        grid=grid,
        in_specs=[
            pl.BlockSpec((B_TILE, N_F), x_idx),
            pl.BlockSpec((N_F, KC), a_idx),
            pl.BlockSpec((KC, N_F), b_idx),
        ],
        out_specs=[
            pl.BlockSpec((B_TILE, N_K), x_idx),
            pl.BlockSpec((B_TILE, N_F), x_idx),
        ],
        out_shape=[
            jax.ShapeDtypeStruct((batch, N_K), jnp.float32),
            jax.ShapeDtypeStruct((batch, N_F), jnp.float32),
        ],
        scratch_shapes=[
            pltpu.VMEM((N_F, N_K), jnp.bfloat16),
            pltpu.VMEM((N_K, N_F), jnp.bfloat16),
        ],
        compiler_params=pltpu.CompilerParams(
            dimension_semantics=("arbitrary",),
        ),
    )(x, A, B)
    return (out, inner)
